# baseline (speedup 1.0000x reference)
import jax
import jax.numpy as jnp
from jax import lax
from jax.experimental import pallas as pl
from jax.experimental.pallas import tpu as pltpu
from jax.experimental.pallas import tpu_sc as plsc

_NC = 1
_NS = 16
_L = 16

_C = 96
_NBUF = 4
_HALF = 25088


def _sc_gather(idx_hbm, x_hbm, out_hbm, idxf_v, sel_v, bufs, gsems, ssems,
               slabs_per_w, hw):
    cid = lax.axis_index("c")
    sid = lax.axis_index("s")
    wid = sid * _NC + cid

    pltpu.sync_copy(idx_hbm, idxf_v)
    zeros = jnp.zeros((_L,), jnp.int32)
    for k in range(_C // _L):
        sel_v[pl.ds(_L * k, _L)] = zeros
    iota = lax.iota(jnp.int32, _L)
    ones = jnp.ones((_L,), jnp.int32)
    offset = zeros
    for k in range(_C // _L):
        v = idxf_v[pl.ds(_L * k, _L)]
        m = v != jnp.zeros((_L,), jnp.float32)
        mi = jnp.where(m, ones, zeros)
        pos = plsc.cumsum(mi) - ones + offset
        plsc.store_scatter(sel_v, [pos], iota + jnp.full((_L,), _L * k, jnp.int32), mask=m)
        offset = offset + lax.broadcast(jnp.sum(mi), (_L,))

    halves_per_slab = hw // _HALF
    n_t = slabs_per_w * halves_per_slab
    base_slab = wid * slabs_per_w

    def src_dst(t):
        slab_local = t // halves_per_slab
        half = t - slab_local * halves_per_slab
        s_global = base_slab + slab_local
        bi = s_global // _C
        j = s_global - bi * _C
        sel_vec = plsc.load_gather(sel_v, [lax.broadcast(j, (_L,))])
        sj = jnp.max(sel_vec)
        src_slab = bi * _C + sj
        off = half * _HALF
        return (x_hbm.at[pl.ds(src_slab, 1), pl.ds(off, _HALF)],
                out_hbm.at[pl.ds(s_global, 1), pl.ds(off, _HALF)])

    def outer(go, carry):
        for b in range(_NBUF):
            t = go * _NBUF + b
            src, _ = src_dst(t)

            @pl.when(go > 0)
            def _():
                pltpu.make_async_copy(
                    bufs[b], out_hbm.at[pl.ds(0, 1), pl.ds(0, _HALF)], ssems[b]
                ).wait()

            pltpu.async_copy(src, bufs[b], gsems[b])
        for b in range(_NBUF):
            t = go * _NBUF + b
            _, dst = src_dst(t)
            pltpu.make_async_copy(
                x_hbm.at[pl.ds(0, 1), pl.ds(0, _HALF)], bufs[b], gsems[b]
            ).wait()
            pltpu.async_copy(bufs[b], dst, ssems[b])
        return carry

    lax.fori_loop(0, n_t // _NBUF, outer, jnp.int32(0))
    for b in range(_NBUF):
        pltpu.make_async_copy(
            bufs[b], out_hbm.at[pl.ds(0, 1), pl.ds(0, _HALF)], ssems[b]
        ).wait()


@jax.jit
def kernel(input_tensor, indexes):
    b, c, h, w = input_tensor.shape
    hw = h * w
    n_slabs = b * c
    n_workers = _NC * _NS
    slabs_per_w = n_slabs // n_workers

    x = input_tensor.reshape(n_slabs, hw)
    mesh = plsc.VectorSubcoreMesh(
        core_axis_name="c", subcore_axis_name="s",
        num_cores=_NC, num_subcores=_NS,
    )

    def body(idx_hbm, x_hbm, out_hbm, idxf_v, sel_v, b0, b1, b2, b3,
             g0, g1, g2, g3, s0, s1, s2, s3):
        _sc_gather(idx_hbm, x_hbm, out_hbm, idxf_v, sel_v,
                   [b0, b1, b2, b3], [g0, g1, g2, g3], [s0, s1, s2, s3],
                   slabs_per_w, hw)

    out = pl.kernel(
        body,
        out_type=jax.ShapeDtypeStruct((n_slabs, hw), jnp.float32),
        mesh=mesh,
        compiler_params=pltpu.CompilerParams(needs_layout_passes=False),
        scratch_types=(
            [pltpu.VMEM((c,), jnp.float32), pltpu.VMEM((c,), jnp.int32)]
            + [pltpu.VMEM((1, _HALF), jnp.float32)] * _NBUF
            + [pltpu.SemaphoreType.DMA] * (2 * _NBUF)
        ),
    )(indexes, x)
    return out.reshape(b, c, h, w)


# speedup vs baseline: 3.3043x; 3.3043x over previous
import jax
import jax.numpy as jnp
from jax import lax
from jax.experimental import pallas as pl
from jax.experimental.pallas import tpu as pltpu
from jax.experimental.pallas import tpu_sc as plsc

_NC = 2
_NS = 16
_L = 16

_C = 96
_NBUF = 2


def _sc_gather(idx_hbm, x_hbm, out_hbm, idxf_v, sel_v, bufs, gsems, ssems,
               slabs_per_w, h, w):
    cid = lax.axis_index("c")
    sid = lax.axis_index("s")
    wid = sid * _NC + cid

    pltpu.sync_copy(idx_hbm, idxf_v)
    zeros = jnp.zeros((_L,), jnp.int32)
    for k in range(_C // _L):
        sel_v[pl.ds(_L * k, _L)] = zeros
    iota = lax.iota(jnp.int32, _L)
    ones = jnp.ones((_L,), jnp.int32)
    offset = zeros
    for k in range(_C // _L):
        v = idxf_v[pl.ds(_L * k, _L)]
        m = v != jnp.zeros((_L,), jnp.float32)
        mi = jnp.where(m, ones, zeros)
        pos = plsc.cumsum(mi) - ones + offset
        plsc.store_scatter(sel_v, [pos], iota + jnp.full((_L,), _L * k, jnp.int32), mask=m)
        offset = offset + lax.broadcast(jnp.sum(mi), (_L,))

    base_slab = wid * slabs_per_w

    def src_dst(t):
        s_global = base_slab + t
        bi = s_global // _C
        j = s_global - bi * _C
        sel_vec = plsc.load_gather(sel_v, [lax.broadcast(j, (_L,))])
        sj = jnp.max(sel_vec)
        return (x_hbm.at[pl.ds(bi, 1), pl.ds(sj, 1)],
                out_hbm.at[pl.ds(bi, 1), pl.ds(j, 1)])

    def outer(go, carry):
        for b in range(_NBUF):
            t = go * _NBUF + b
            src, _ = src_dst(t)

            @pl.when(go > 0)
            def _():
                pltpu.make_async_copy(
                    bufs[b], out_hbm.at[pl.ds(0, 1), pl.ds(0, 1)], ssems[b]
                ).wait()

            pltpu.async_copy(src, bufs[b], gsems[b])
        for b in range(_NBUF):
            t = go * _NBUF + b
            _, dst = src_dst(t)
            pltpu.make_async_copy(
                x_hbm.at[pl.ds(0, 1), pl.ds(0, 1)], bufs[b], gsems[b]
            ).wait()
            pltpu.async_copy(bufs[b], dst, ssems[b])
        return carry

    lax.fori_loop(0, slabs_per_w // _NBUF, outer, jnp.int32(0))
    for b in range(_NBUF):
        pltpu.make_async_copy(
            bufs[b], out_hbm.at[pl.ds(0, 1), pl.ds(0, 1)], ssems[b]
        ).wait()


@jax.jit
def kernel(input_tensor, indexes):
    b, c, h, w = input_tensor.shape
    n_slabs = b * c
    n_workers = _NC * _NS
    slabs_per_w = n_slabs // n_workers

    mesh = plsc.VectorSubcoreMesh(
        core_axis_name="c", subcore_axis_name="s",
        num_cores=_NC, num_subcores=_NS,
    )

    def body(idx_hbm, x_hbm, out_hbm, idxf_v, sel_v, b0, b1, g0, g1, s0, s1):
        _sc_gather(idx_hbm, x_hbm, out_hbm, idxf_v, sel_v,
                   [b0, b1], [g0, g1], [s0, s1], slabs_per_w, h, w)

    return pl.kernel(
        body,
        out_type=jax.ShapeDtypeStruct((b, c, h, w), jnp.float32),
        mesh=mesh,
        compiler_params=pltpu.CompilerParams(
            needs_layout_passes=False, use_tc_tiling_on_sc=True,
        ),
        scratch_types=(
            [pltpu.VMEM((c,), jnp.float32), pltpu.VMEM((c,), jnp.int32)]
            + [pltpu.VMEM((1, 1, h, w), jnp.float32)] * _NBUF
            + [pltpu.SemaphoreType.DMA] * (2 * _NBUF)
        ),
    )(indexes, input_tensor)


# speedup vs baseline: 3.4398x; 1.0410x over previous
import jax
import jax.numpy as jnp
from jax import lax
from jax.experimental import pallas as pl
from jax.experimental.pallas import tpu as pltpu
from jax.experimental.pallas import tpu_sc as plsc

_NC = 2
_NS = 16
_L = 16

_C = 96
_NBUF = 4


def _sc_gather(idx_hbm, x_hbm, out_hbm, idxf_v, sel_v, bufs, gsems, ssems,
               slabs_per_w, h, w):
    cid = lax.axis_index("c")
    sid = lax.axis_index("s")
    wid = sid * _NC + cid

    pltpu.sync_copy(idx_hbm, idxf_v)
    zeros = jnp.zeros((_L,), jnp.int32)
    for k in range(_C // _L):
        sel_v[pl.ds(_L * k, _L)] = zeros
    iota = lax.iota(jnp.int32, _L)
    ones = jnp.ones((_L,), jnp.int32)
    offset = zeros
    for k in range(_C // _L):
        v = idxf_v[pl.ds(_L * k, _L)]
        m = v != jnp.zeros((_L,), jnp.float32)
        mi = jnp.where(m, ones, zeros)
        pos = plsc.cumsum(mi) - ones + offset
        plsc.store_scatter(sel_v, [pos], iota + jnp.full((_L,), _L * k, jnp.int32), mask=m)
        offset = offset + lax.broadcast(jnp.sum(mi), (_L,))

    base_slab = wid * slabs_per_w
    hh = h // 2

    def src_dst(t):
        s_global = base_slab + t // 2
        half = t - (t // 2) * 2
        bi = s_global // _C
        j = s_global - bi * _C
        sel_vec = plsc.load_gather(sel_v, [lax.broadcast(j, (_L,))])
        sj = jnp.max(sel_vec)
        ro = half * hh
        return (x_hbm.at[pl.ds(bi, 1), pl.ds(sj, 1), pl.ds(ro, hh)],
                out_hbm.at[pl.ds(bi, 1), pl.ds(j, 1), pl.ds(ro, hh)])

    def outer(go, carry):
        for b in range(_NBUF):
            t = go * _NBUF + b
            src, _ = src_dst(t)

            @pl.when(go > 0)
            def _():
                pltpu.make_async_copy(
                    bufs[b], out_hbm.at[pl.ds(0, 1), pl.ds(0, 1), pl.ds(0, h // 2)], ssems[b]
                ).wait()

            pltpu.async_copy(src, bufs[b], gsems[b])
        for b in range(_NBUF):
            t = go * _NBUF + b
            _, dst = src_dst(t)
            pltpu.make_async_copy(
                x_hbm.at[pl.ds(0, 1), pl.ds(0, 1), pl.ds(0, h // 2)], bufs[b], gsems[b]
            ).wait()
            pltpu.async_copy(bufs[b], dst, ssems[b])
        return carry

    lax.fori_loop(0, 2 * slabs_per_w // _NBUF, outer, jnp.int32(0))
    for b in range(_NBUF):
        pltpu.make_async_copy(
            bufs[b], out_hbm.at[pl.ds(0, 1), pl.ds(0, 1), pl.ds(0, h // 2)], ssems[b]
        ).wait()


@jax.jit
def kernel(input_tensor, indexes):
    b, c, h, w = input_tensor.shape
    n_slabs = b * c
    n_workers = _NC * _NS
    slabs_per_w = n_slabs // n_workers

    mesh = plsc.VectorSubcoreMesh(
        core_axis_name="c", subcore_axis_name="s",
        num_cores=_NC, num_subcores=_NS,
    )

    def body(idx_hbm, x_hbm, out_hbm, idxf_v, sel_v, b0, b1, b2, b3,
             g0, g1, g2, g3, s0, s1, s2, s3):
        _sc_gather(idx_hbm, x_hbm, out_hbm, idxf_v, sel_v,
                   [b0, b1, b2, b3], [g0, g1, g2, g3], [s0, s1, s2, s3],
                   slabs_per_w, h, w)

    return pl.kernel(
        body,
        out_type=jax.ShapeDtypeStruct((b, c, h, w), jnp.float32),
        mesh=mesh,
        compiler_params=pltpu.CompilerParams(
            needs_layout_passes=False, use_tc_tiling_on_sc=True,
        ),
        scratch_types=(
            [pltpu.VMEM((c,), jnp.float32), pltpu.VMEM((c,), jnp.int32)]
            + [pltpu.VMEM((1, 1, h // 2, w), jnp.float32)] * _NBUF
            + [pltpu.SemaphoreType.DMA] * (2 * _NBUF)
        ),
    )(indexes, input_tensor)


# speedup vs baseline: 3.4927x; 1.0154x over previous
import jax
import jax.numpy as jnp
from jax import lax
from jax.experimental import pallas as pl
from jax.experimental.pallas import tpu as pltpu
from jax.experimental.pallas import tpu_sc as plsc

_NC = 2
_NS = 16
_L = 16

_C = 96
_NBUF = 8


def _sc_gather(idx_hbm, x_hbm, out_hbm, idxf_v, sel_v, bufs, gsems, ssems,
               slabs_per_w, h, w):
    cid = lax.axis_index("c")
    sid = lax.axis_index("s")
    wid = sid * _NC + cid

    pltpu.sync_copy(idx_hbm, idxf_v)
    zeros = jnp.zeros((_L,), jnp.int32)
    for k in range(_C // _L):
        sel_v[pl.ds(_L * k, _L)] = zeros
    iota = lax.iota(jnp.int32, _L)
    ones = jnp.ones((_L,), jnp.int32)
    offset = zeros
    for k in range(_C // _L):
        v = idxf_v[pl.ds(_L * k, _L)]
        m = v != jnp.zeros((_L,), jnp.float32)
        mi = jnp.where(m, ones, zeros)
        pos = plsc.cumsum(mi) - ones + offset
        plsc.store_scatter(sel_v, [pos], iota + jnp.full((_L,), _L * k, jnp.int32), mask=m)
        offset = offset + lax.broadcast(jnp.sum(mi), (_L,))

    base_slab = wid * slabs_per_w
    hh = h // 4

    def src_dst(t):
        s_global = base_slab + t // 4
        half = t - (t // 4) * 4
        bi = s_global // _C
        j = s_global - bi * _C
        sel_vec = plsc.load_gather(sel_v, [lax.broadcast(j, (_L,))])
        sj = jnp.max(sel_vec)
        ro = half * hh
        return (x_hbm.at[pl.ds(bi, 1), pl.ds(sj, 1), pl.ds(ro, hh)],
                out_hbm.at[pl.ds(bi, 1), pl.ds(j, 1), pl.ds(ro, hh)])

    def outer(go, carry):
        for b in range(_NBUF):
            t = go * _NBUF + b
            src, _ = src_dst(t)

            @pl.when(go > 0)
            def _():
                pltpu.make_async_copy(
                    bufs[b], out_hbm.at[pl.ds(0, 1), pl.ds(0, 1), pl.ds(0, h // 4)], ssems[b]
                ).wait()

            pltpu.async_copy(src, bufs[b], gsems[b])
        for b in range(_NBUF):
            t = go * _NBUF + b
            _, dst = src_dst(t)
            pltpu.make_async_copy(
                x_hbm.at[pl.ds(0, 1), pl.ds(0, 1), pl.ds(0, h // 4)], bufs[b], gsems[b]
            ).wait()
            pltpu.async_copy(bufs[b], dst, ssems[b])
        return carry

    lax.fori_loop(0, 4 * slabs_per_w // _NBUF, outer, jnp.int32(0))
    for b in range(_NBUF):
        pltpu.make_async_copy(
            bufs[b], out_hbm.at[pl.ds(0, 1), pl.ds(0, 1), pl.ds(0, h // 4)], ssems[b]
        ).wait()


@jax.jit
def kernel(input_tensor, indexes):
    b, c, h, w = input_tensor.shape
    n_slabs = b * c
    n_workers = _NC * _NS
    slabs_per_w = n_slabs // n_workers

    mesh = plsc.VectorSubcoreMesh(
        core_axis_name="c", subcore_axis_name="s",
        num_cores=_NC, num_subcores=_NS,
    )

    def body(idx_hbm, x_hbm, out_hbm, idxf_v, sel_v, *rest):
        _sc_gather(idx_hbm, x_hbm, out_hbm, idxf_v, sel_v,
                   list(rest[0:_NBUF]), list(rest[_NBUF:2 * _NBUF]),
                   list(rest[2 * _NBUF:3 * _NBUF]), slabs_per_w, h, w)

    return pl.kernel(
        body,
        out_type=jax.ShapeDtypeStruct((b, c, h, w), jnp.float32),
        mesh=mesh,
        compiler_params=pltpu.CompilerParams(
            needs_layout_passes=False, use_tc_tiling_on_sc=True,
        ),
        scratch_types=(
            [pltpu.VMEM((c,), jnp.float32), pltpu.VMEM((c,), jnp.int32)]
            + [pltpu.VMEM((1, 1, h // 4, w), jnp.float32)] * _NBUF
            + [pltpu.SemaphoreType.DMA] * (2 * _NBUF)
        ),
    )(indexes, input_tensor)
